# baseline (speedup 1.0000x reference)
import functools

import jax
import jax.numpy as jnp
from jax import lax
from jax.experimental import pallas as pl
from jax.experimental.pallas import tpu as pltpu
from jax.experimental.pallas import tpu_sc as plsc

N_NODES = 10000
R_ACC = 10240
STRIPE = R_ACC // 16
CHUNK = 64
NC, NS = 2, 16
NW = NC * NS
SB = 16
NB = 4
LOOKAHEAD = 2


def _sc_aggregate_body(x_hbm, src_hbm, dst_hbm, zeros_hbm, out_hbm,
                       src_v, dst_v, msg_v, agg, gsems, ssems, k_chunks):
    cid = lax.axis_index("c")
    sid = lax.axis_index("s")
    wid = sid * NC + cid

    pltpu.sync_copy(zeros_hbm, agg.at[pl.ds(sid * STRIPE, STRIPE)])
    plsc.subcore_barrier()

    def slab_load(j, slot):
        off = wid * k_chunks + j * SB
        pltpu.sync_copy(src_hbm.at[pl.ds(off, SB)], src_v.at[slot])
        pltpu.sync_copy(dst_hbm.at[pl.ds(off, SB)], dst_v.at[slot])

    def idx_ref(ref, c):
        return ref.at[(c // SB) % 2, c % SB]

    def fire_gather(c, b):
        pltpu.async_copy(x_hbm.at[idx_ref(src_v, c)], msg_v.at[b], gsems[b])

    def wait_gather(c, b):
        pltpu.make_async_copy(x_hbm.at[idx_ref(src_v, c)],
                              msg_v.at[b], gsems[b]).wait()

    def fire_scatter(c, b):
        pltpu.async_copy(msg_v.at[b], agg.at[idx_ref(dst_v, c)],
                         ssems[b], add=True)

    def wait_scatter(c, b):
        pltpu.make_async_copy(msg_v.at[b], agg.at[idx_ref(dst_v, c)],
                              ssems[b]).wait()

    n_quads = k_chunks // NB
    slab_load(0, 0)
    for c in range(LOOKAHEAD):
        fire_gather(c, c % NB)

    def quad(q, carry):
        for u in range(NB):
            c = q * NB + u
            b = u
            wait_gather(c, b)
            fire_scatter(c, b)

            @pl.when(c >= NB - LOOKAHEAD)
            def _():
                wait_scatter(c - (NB - LOOKAHEAD), (b + LOOKAHEAD) % NB)

            cg = c + LOOKAHEAD

            @pl.when(jnp.logical_and(cg % SB == 0, cg < k_chunks))
            def _():
                slab_load(cg // SB, (cg // SB) % 2)

            @pl.when(cg < k_chunks)
            def _():
                fire_gather(cg, (b + LOOKAHEAD) % NB)

        return carry

    lax.fori_loop(0, n_quads, quad, 0)
    for c in range(k_chunks - (NB - LOOKAHEAD), k_chunks):
        wait_scatter(c, c % NB)
    plsc.subcore_barrier()

    off = cid * R_ACC + sid * STRIPE
    pltpu.sync_copy(agg.at[pl.ds(sid * STRIPE, STRIPE)],
                    out_hbm.at[pl.ds(off, STRIPE)])


def _sc_aggregate(x, src2, dst2, zeros, k_chunks):
    mesh = plsc.VectorSubcoreMesh(core_axis_name="c", subcore_axis_name="s",
                                  num_cores=NC, num_subcores=NS)
    body = functools.partial(_sc_aggregate_body, k_chunks=k_chunks)
    return pl.kernel(
        body,
        out_type=jax.ShapeDtypeStruct((NC * R_ACC, 128), jnp.float32),
        mesh=mesh,
        scratch_types=[
            pltpu.VMEM((2, SB, CHUNK), jnp.int32),
            pltpu.VMEM((2, SB, CHUNK), jnp.int32),
            pltpu.VMEM((NB, CHUNK, 128), jnp.float32),
            pltpu.VMEM_SHARED((R_ACC, 128), jnp.float32),
            [pltpu.SemaphoreType.DMA] * NB,
            [pltpu.SemaphoreType.DMA] * NB,
        ],
    )(x, src2, dst2, zeros)


def _tc_head_body(p_ref, w_ref, b_ref, mw_ref, mb_ref, o_ref):
    s = p_ref[0] + p_ref[1]
    h = jnp.dot(s, w_ref[...], preferred_element_type=jnp.float32,
                precision=lax.Precision.HIGHEST)
    h = jnp.maximum(h + b_ref[...], 0.0)
    o = lax.dot_general(h, mw_ref[...], (((1,), (1,)), ((), ())),
                        preferred_element_type=jnp.float32,
                        precision=lax.Precision.HIGHEST)
    o_ref[...] = o + mb_ref[...]


def _tc_head(partials, W, b, mlp_W, mlp_b):
    blk = 400
    grid = (N_NODES // blk,)
    return pl.pallas_call(
        _tc_head_body,
        grid=grid,
        in_specs=[
            pl.BlockSpec((2, blk, 128), lambda i: (0, i, 0)),
            pl.BlockSpec((128, 128), lambda i: (0, 0)),
            pl.BlockSpec((1, 128), lambda i: (0, 0)),
            pl.BlockSpec((64, 128), lambda i: (0, 0)),
            pl.BlockSpec((1, 64), lambda i: (0, 0)),
        ],
        out_specs=pl.BlockSpec((blk, 64), lambda i: (i, 0)),
        out_shape=jax.ShapeDtypeStruct((N_NODES, 64), jnp.float32),
    )(partials, W, b, mlp_W, mlp_b)


def kernel(x, adj, W, b, mlp_W, mlp_b):
    src = adj[0]
    dst = adj[1]
    e = src.shape[0]
    k_chunks = -(-e // (CHUNK * NW))
    k_chunks = -(-k_chunks // 8) * 8
    e_pad = NW * k_chunks * CHUNK
    pad = e_pad - e
    pad_i = jnp.arange(pad, dtype=jnp.int32)
    src_p = jnp.concatenate([src, pad_i % N_NODES])
    dst_p = jnp.concatenate([dst, N_NODES + pad_i % (R_ACC - N_NODES)])
    src2 = src_p.reshape(NW * k_chunks, CHUNK)
    dst2 = dst_p.reshape(NW * k_chunks, CHUNK)
    zeros = jnp.zeros((STRIPE, 128), jnp.float32)

    partials = _sc_aggregate(x, src2, dst2, zeros, k_chunks)
    partials = partials.reshape(NC, R_ACC, 128)
    return _tc_head(partials, W, b.reshape(1, 128), mlp_W, mlp_b.reshape(1, 64))


# speedup vs baseline: 1.0291x; 1.0291x over previous
import jax
import jax.numpy as jnp
from jax import lax
from jax.experimental import pallas as pl
from jax.experimental.pallas import tpu as pltpu
from jax.experimental.pallas import tpu_sc as plsc

N_NODES = 10000
R_ACC = 10240
STRIPE = R_ACC // 16
CHUNK = 64
NC, NS = 2, 16
NW = NC * NS
SB = 8
NB = 4
LOOKAHEAD = 2


def _sc_aggregate_body(x_hbm, adj_hbm, out_hbm, src_v, dst_v, msg_v, agg,
                       gsems, ssems):
    cid = lax.axis_index("c")
    sid = lax.axis_index("s")
    wid = sid * NC + cid

    n_chunks = adj_hbm.shape[1]
    n_blocks = n_chunks // SB
    base_blocks = n_blocks // NW
    rem_blocks = n_blocks - NW * base_blocks
    start = (wid * base_blocks + jnp.minimum(wid, rem_blocks)) * SB
    k_w = (base_blocks + jnp.where(wid < rem_blocks, 1, 0)) * SB

    def zrow(r, carry):
        for col in range(128 // 16):
            msg_v[0, r, pl.ds(col * 16, 16)] = jnp.zeros((16,), jnp.float32)
        return carry

    lax.fori_loop(0, CHUNK, zrow, 0)

    def zcopy(j, carry):
        pltpu.sync_copy(msg_v.at[0], agg.at[pl.ds(sid * STRIPE + j * CHUNK,
                                                  CHUNK)])
        return carry

    lax.fori_loop(0, STRIPE // CHUNK, zcopy, 0)
    plsc.subcore_barrier()

    def slab_load(j, slot):
        off = start + j * SB
        pltpu.sync_copy(adj_hbm.at[0, pl.ds(off, SB)], src_v.at[slot])
        pltpu.sync_copy(adj_hbm.at[1, pl.ds(off, SB)], dst_v.at[slot])

    def idx_ref(ref, c):
        return ref.at[(c // SB) % 2, c % SB]

    def fire_gather(c, b):
        pltpu.async_copy(x_hbm.at[idx_ref(src_v, c)], msg_v.at[b], gsems[b])

    def wait_gather(c, b):
        pltpu.make_async_copy(x_hbm.at[idx_ref(src_v, c)],
                              msg_v.at[b], gsems[b]).wait()

    def fire_scatter(c, b):
        pltpu.async_copy(msg_v.at[b], agg.at[idx_ref(dst_v, c)],
                         ssems[b], add=True)

    def wait_scatter(c, b):
        pltpu.make_async_copy(msg_v.at[b], agg.at[idx_ref(dst_v, c)],
                              ssems[b]).wait()

    slab_load(0, 0)
    for c in range(LOOKAHEAD):
        fire_gather(c, c % NB)

    def quad(q, carry):
        for u in range(NB):
            c = q * NB + u
            b = u
            wait_gather(c, b)
            fire_scatter(c, b)

            @pl.when(c >= NB - LOOKAHEAD)
            def _():
                wait_scatter(c - (NB - LOOKAHEAD), (b + LOOKAHEAD) % NB)

            cg = c + LOOKAHEAD

            @pl.when(jnp.logical_and(cg % SB == 0, cg < k_w))
            def _():
                slab_load(cg // SB, (cg // SB) % 2)

            @pl.when(cg < k_w)
            def _():
                fire_gather(cg, (b + LOOKAHEAD) % NB)

        return carry

    lax.fori_loop(0, k_w // NB, quad, 0)
    wait_scatter(k_w - 2, NB - 2)
    wait_scatter(k_w - 1, NB - 1)
    plsc.subcore_barrier()

    off = cid * R_ACC + sid * STRIPE
    pltpu.sync_copy(agg.at[pl.ds(sid * STRIPE, STRIPE)],
                    out_hbm.at[pl.ds(off, STRIPE)])


def _sc_aggregate(x, adj3):
    mesh = plsc.VectorSubcoreMesh(core_axis_name="c", subcore_axis_name="s",
                                  num_cores=NC, num_subcores=NS)
    return pl.kernel(
        _sc_aggregate_body,
        out_type=jax.ShapeDtypeStruct((NC * R_ACC, 128), jnp.float32),
        mesh=mesh,
        scratch_types=[
            pltpu.VMEM((2, SB, CHUNK), jnp.int32),
            pltpu.VMEM((2, SB, CHUNK), jnp.int32),
            pltpu.VMEM((NB, CHUNK, 128), jnp.float32),
            pltpu.VMEM_SHARED((R_ACC, 128), jnp.float32),
            [pltpu.SemaphoreType.DMA] * NB,
            [pltpu.SemaphoreType.DMA] * NB,
        ],
    )(x, adj3)


def _tc_head_body(p_ref, w_ref, b_ref, mw_ref, mb_ref, o_ref):
    s = p_ref[0] + p_ref[1]
    h = jnp.dot(s, w_ref[...], preferred_element_type=jnp.float32,
                precision=lax.Precision.HIGHEST)
    h = jnp.maximum(h + b_ref[...], 0.0)
    o = lax.dot_general(h, mw_ref[...], (((1,), (1,)), ((), ())),
                        preferred_element_type=jnp.float32,
                        precision=lax.Precision.HIGHEST)
    o_ref[...] = o + mb_ref[...]


def _tc_head(partials, W, b, mlp_W, mlp_b):
    blk = 400
    grid = (N_NODES // blk,)
    return pl.pallas_call(
        _tc_head_body,
        grid=grid,
        in_specs=[
            pl.BlockSpec((2, blk, 128), lambda i: (0, i, 0)),
            pl.BlockSpec((128, 128), lambda i: (0, 0)),
            pl.BlockSpec((1, 128), lambda i: (0, 0)),
            pl.BlockSpec((64, 128), lambda i: (0, 0)),
            pl.BlockSpec((1, 64), lambda i: (0, 0)),
        ],
        out_specs=pl.BlockSpec((blk, 64), lambda i: (i, 0)),
        out_shape=jax.ShapeDtypeStruct((N_NODES, 64), jnp.float32),
    )(partials, W, b, mlp_W, mlp_b)


def kernel(x, adj, W, b, mlp_W, mlp_b):
    e = adj.shape[1]
    adj3 = adj.reshape(2, e // CHUNK, CHUNK)
    partials = _sc_aggregate(x, adj3)
    partials = partials.reshape(NC, R_ACC, 128)
    return _tc_head(partials, W, b.reshape(1, 128), mlp_W, mlp_b.reshape(1, 64))


# speedup vs baseline: 1.0713x; 1.0410x over previous
import jax
import jax.numpy as jnp
from jax import lax
from jax.experimental import pallas as pl
from jax.experimental.pallas import tpu as pltpu
from jax.experimental.pallas import tpu_sc as plsc

N_NODES = 10000
R_ACC = 10240
STRIPE = R_ACC // 16
CHUNK = 64
NC, NS = 2, 16
NW = NC * NS
SB = 8
NB = 4
LOOKAHEAD = 2


def _sc_aggregate_body(x_hbm, adj_hbm, out_hbm, src_v, dst_v, msg_v, agg,
                       gsems, ssems):
    cid = lax.axis_index("c")
    sid = lax.axis_index("s")
    wid = sid * NC + cid

    n_chunks = adj_hbm.shape[1]
    n_blocks = n_chunks // SB
    base_blocks = n_blocks // NW
    rem_blocks = n_blocks - NW * base_blocks
    start = (wid * base_blocks + jnp.minimum(wid, rem_blocks)) * SB
    k_w = (base_blocks + jnp.where(wid < rem_blocks, 1, 0)) * SB

    def zrow(r, carry):
        for col in range(128 // 16):
            msg_v[0, r, pl.ds(col * 16, 16)] = jnp.zeros((16,), jnp.float32)
        return carry

    lax.fori_loop(0, CHUNK, zrow, 0)

    def zcopy(j, carry):
        pltpu.sync_copy(msg_v.at[0], agg.at[pl.ds(sid * STRIPE + j * CHUNK,
                                                  CHUNK)])
        return carry

    lax.fori_loop(0, STRIPE // CHUNK, zcopy, 0)
    plsc.subcore_barrier()

    def slab_load(j, slot):
        off = start + j * SB
        pltpu.sync_copy(adj_hbm.at[0, pl.ds(off, SB)], src_v.at[slot])
        pltpu.sync_copy(adj_hbm.at[1, pl.ds(off, SB)], dst_v.at[slot])

    def idx_ref(ref, c):
        return ref.at[(c // SB) % 2, c % SB]

    def fire_gather(c, b):
        pltpu.async_copy(x_hbm.at[idx_ref(src_v, c)], msg_v.at[b], gsems[b])

    def wait_gather(c, b):
        pltpu.make_async_copy(x_hbm.at[idx_ref(src_v, c)],
                              msg_v.at[b], gsems[b]).wait()

    def fire_scatter(c, b):
        pltpu.async_copy(msg_v.at[b], agg.at[idx_ref(dst_v, c)],
                         ssems[b], add=True)

    def wait_scatter(c, b):
        pltpu.make_async_copy(msg_v.at[b], agg.at[idx_ref(dst_v, c)],
                              ssems[b]).wait()

    slab_load(0, 0)
    for c in range(LOOKAHEAD):
        fire_gather(c, c % NB)

    def quad(q, carry):
        for u in range(NB):
            c = q * NB + u
            b = u
            wait_gather(c, b)
            fire_scatter(c, b)

            @pl.when(c >= NB - LOOKAHEAD)
            def _():
                wait_scatter(c - (NB - LOOKAHEAD), (b + LOOKAHEAD) % NB)

            cg = c + LOOKAHEAD

            @pl.when(jnp.logical_and(cg % SB == 0, cg < k_w))
            def _():
                slab_load(cg // SB, (cg // SB) % 2)

            @pl.when(cg < k_w)
            def _():
                fire_gather(cg, (b + LOOKAHEAD) % NB)

        return carry

    lax.fori_loop(0, k_w // NB, quad, 0)
    wait_scatter(k_w - 2, NB - 2)
    wait_scatter(k_w - 1, NB - 1)
    plsc.subcore_barrier()

    off = cid * R_ACC + sid * STRIPE
    pltpu.sync_copy(agg.at[pl.ds(sid * STRIPE, STRIPE)],
                    out_hbm.at[pl.ds(off, STRIPE)])


def _sc_aggregate(x, adj3):
    mesh = plsc.VectorSubcoreMesh(core_axis_name="c", subcore_axis_name="s",
                                  num_cores=NC, num_subcores=NS)
    return pl.kernel(
        _sc_aggregate_body,
        out_type=jax.ShapeDtypeStruct((NC * R_ACC, 128), jnp.float32),
        mesh=mesh,
        scratch_types=[
            pltpu.VMEM((2, SB, CHUNK), jnp.int32),
            pltpu.VMEM((2, SB, CHUNK), jnp.int32),
            pltpu.VMEM((NB, CHUNK, 128), jnp.float32),
            pltpu.VMEM_SHARED((R_ACC, 128), jnp.float32),
            [pltpu.SemaphoreType.DMA] * NB,
            [pltpu.SemaphoreType.DMA] * NB,
        ],
    )(x, adj3)


def _tc_head_body(p_ref, w_ref, b_ref, mw_ref, mb_ref, o_ref):
    s = p_ref[0] + p_ref[1]
    h = jnp.dot(s, w_ref[...], preferred_element_type=jnp.float32,
                precision=lax.Precision.HIGHEST)
    h = jnp.maximum(h + b_ref[...], 0.0)
    o = lax.dot_general(h, mw_ref[...], (((1,), (1,)), ((), ())),
                        preferred_element_type=jnp.float32,
                        precision=lax.Precision.HIGHEST)
    o_ref[...] = o + mb_ref[...]


def _tc_head(partials, W, b, mlp_W, mlp_b):
    blk = 2000
    grid = (N_NODES // blk,)
    return pl.pallas_call(
        _tc_head_body,
        grid=grid,
        in_specs=[
            pl.BlockSpec((2, blk, 128), lambda i: (0, i, 0)),
            pl.BlockSpec((128, 128), lambda i: (0, 0)),
            pl.BlockSpec((1, 128), lambda i: (0, 0)),
            pl.BlockSpec((64, 128), lambda i: (0, 0)),
            pl.BlockSpec((1, 64), lambda i: (0, 0)),
        ],
        out_specs=pl.BlockSpec((blk, 64), lambda i: (i, 0)),
        out_shape=jax.ShapeDtypeStruct((N_NODES, 64), jnp.float32),
    )(partials, W, b, mlp_W, mlp_b)


def kernel(x, adj, W, b, mlp_W, mlp_b):
    e = adj.shape[1]
    adj3 = adj.reshape(2, e // CHUNK, CHUNK)
    partials = _sc_aggregate(x, adj3)
    partials = partials.reshape(NC, R_ACC, 128)
    return _tc_head(partials, W, b.reshape(1, 128), mlp_W, mlp_b.reshape(1, 64))


# speedup vs baseline: 1.2078x; 1.1274x over previous
import jax
import jax.numpy as jnp
from jax import lax
from jax.experimental import pallas as pl
from jax.experimental.pallas import tpu as pltpu
from jax.experimental.pallas import tpu_sc as plsc

N_NODES = 10000
R_ACC = 10240
STRIPE = R_ACC // 16
CHUNK = 64
NC, NS = 2, 16
NW = NC * NS
SB = 8
NB = 4
LOOKAHEAD = 2


def _sc_aggregate_body(x_hbm, adj_hbm, out_hbm, src_v, dst_v, msg_v, agg,
                       gsems, ssems, slsem):
    cid = lax.axis_index("c")
    sid = lax.axis_index("s")
    wid = sid * NC + cid

    n_chunks = adj_hbm.shape[1]
    n_blocks = n_chunks // SB
    base_blocks = n_blocks // NW
    rem_blocks = n_blocks - NW * base_blocks
    start = (wid * base_blocks + jnp.minimum(wid, rem_blocks)) * SB
    k_w = (base_blocks + jnp.where(wid < rem_blocks, 1, 0)) * SB

    def zrow(r, carry):
        for col in range(128 // 16):
            msg_v[0, r, pl.ds(col * 16, 16)] = jnp.zeros((16,), jnp.float32)
        return carry

    lax.fori_loop(0, CHUNK, zrow, 0)

    def zcopy(j, carry):
        pltpu.sync_copy(msg_v.at[0], agg.at[pl.ds(sid * STRIPE + j * CHUNK,
                                                  CHUNK)])
        return carry

    lax.fori_loop(0, STRIPE // CHUNK, zcopy, 0)
    plsc.subcore_barrier()

    def slab_load(j, slot):
        off = start + j * SB
        pltpu.sync_copy(adj_hbm.at[0, pl.ds(off, SB)], src_v.at[slot])
        pltpu.sync_copy(adj_hbm.at[1, pl.ds(off, SB)], dst_v.at[slot])

    def fire_slab(j, slot):
        off = start + j * SB
        pltpu.async_copy(adj_hbm.at[0, pl.ds(off, SB)], src_v.at[slot], slsem)
        pltpu.async_copy(adj_hbm.at[1, pl.ds(off, SB)], dst_v.at[slot], slsem)

    def wait_slab(j, slot):
        off = start + j * SB
        pltpu.make_async_copy(adj_hbm.at[0, pl.ds(off, SB)],
                              src_v.at[slot], slsem).wait()
        pltpu.make_async_copy(adj_hbm.at[1, pl.ds(off, SB)],
                              dst_v.at[slot], slsem).wait()

    def idx_ref(ref, c):
        return ref.at[(c // SB) % 2, c % SB]

    def fire_gather(c, b):
        pltpu.async_copy(x_hbm.at[idx_ref(src_v, c)], msg_v.at[b], gsems[b])

    def wait_gather(c, b):
        pltpu.make_async_copy(x_hbm.at[idx_ref(src_v, c)],
                              msg_v.at[b], gsems[b]).wait()

    def fire_scatter(c, b):
        pltpu.async_copy(msg_v.at[b], agg.at[idx_ref(dst_v, c)],
                         ssems[b], add=True)

    def wait_scatter(c, b):
        pltpu.make_async_copy(msg_v.at[b], agg.at[idx_ref(dst_v, c)],
                              ssems[b]).wait()

    slab_load(0, 0)
    for c in range(LOOKAHEAD):
        fire_gather(c, c % NB)

    def quad(q, carry):
        for u in range(NB):
            c = q * NB + u
            b = u
            wait_gather(c, b)
            fire_scatter(c, b)

            @pl.when(c >= NB - LOOKAHEAD)
            def _():
                wait_scatter(c - (NB - LOOKAHEAD), (b + LOOKAHEAD) % NB)

            cg = c + LOOKAHEAD

            @pl.when(jnp.logical_and(c % SB == 2, c < k_w - SB))
            def _():
                jn = c // SB + 1
                fire_slab(jn, jn % 2)

            @pl.when(jnp.logical_and(cg % SB == 0, cg < k_w))
            def _():
                wait_slab(cg // SB, (cg // SB) % 2)

            @pl.when(cg < k_w)
            def _():
                fire_gather(cg, (b + LOOKAHEAD) % NB)

        return carry

    lax.fori_loop(0, k_w // NB, quad, 0)
    wait_scatter(k_w - 2, NB - 2)
    wait_scatter(k_w - 1, NB - 1)
    plsc.subcore_barrier()

    off = cid * R_ACC + sid * STRIPE
    pltpu.sync_copy(agg.at[pl.ds(sid * STRIPE, STRIPE)],
                    out_hbm.at[pl.ds(off, STRIPE)])


def _sc_aggregate(x, adj3):
    mesh = plsc.VectorSubcoreMesh(core_axis_name="c", subcore_axis_name="s",
                                  num_cores=NC, num_subcores=NS)
    return pl.kernel(
        _sc_aggregate_body,
        out_type=jax.ShapeDtypeStruct((NC * R_ACC, 128), jnp.float32),
        mesh=mesh,
        scratch_types=[
            pltpu.VMEM((2, SB, CHUNK), jnp.int32),
            pltpu.VMEM((2, SB, CHUNK), jnp.int32),
            pltpu.VMEM((NB, CHUNK, 128), jnp.float32),
            pltpu.VMEM_SHARED((R_ACC, 128), jnp.float32),
            [pltpu.SemaphoreType.DMA] * NB,
            [pltpu.SemaphoreType.DMA] * NB,
            pltpu.SemaphoreType.DMA,
        ],
    )(x, adj3)


def _tc_head_body(p_ref, w_ref, b_ref, mw_ref, mb_ref, o_ref):
    s = p_ref[0] + p_ref[1]
    h = jnp.dot(s, w_ref[...], preferred_element_type=jnp.float32,
                precision=lax.Precision.HIGHEST)
    h = jnp.maximum(h + b_ref[...], 0.0)
    o = lax.dot_general(h, mw_ref[...], (((1,), (1,)), ((), ())),
                        preferred_element_type=jnp.float32,
                        precision=lax.Precision.HIGHEST)
    o_ref[...] = o + mb_ref[...]


def _tc_head(partials, W, b, mlp_W, mlp_b):
    blk = 2000
    grid = (N_NODES // blk,)
    return pl.pallas_call(
        _tc_head_body,
        grid=grid,
        in_specs=[
            pl.BlockSpec((2, blk, 128), lambda i: (0, i, 0)),
            pl.BlockSpec((128, 128), lambda i: (0, 0)),
            pl.BlockSpec((1, 128), lambda i: (0, 0)),
            pl.BlockSpec((64, 128), lambda i: (0, 0)),
            pl.BlockSpec((1, 64), lambda i: (0, 0)),
        ],
        out_specs=pl.BlockSpec((blk, 64), lambda i: (i, 0)),
        out_shape=jax.ShapeDtypeStruct((N_NODES, 64), jnp.float32),
    )(partials, W, b, mlp_W, mlp_b)


def kernel(x, adj, W, b, mlp_W, mlp_b):
    e = adj.shape[1]
    adj3 = adj.reshape(2, e // CHUNK, CHUNK)
    partials = _sc_aggregate(x, adj3)
    partials = partials.reshape(NC, R_ACC, 128)
    return _tc_head(partials, W, b.reshape(1, 128), mlp_W, mlp_b.reshape(1, 64))


# speedup vs baseline: 1.3488x; 1.1168x over previous
import jax
import jax.numpy as jnp
from jax import lax
from jax.experimental import pallas as pl
from jax.experimental.pallas import tpu as pltpu
from jax.experimental.pallas import tpu_sc as plsc

N_NODES = 10000
R_ACC = 10240
STRIPE = R_ACC // 16
CHUNK = 64
NC, NS = 2, 16
NW = NC * NS
SB = 8
NB = 4
LOOKAHEAD = 3


def _sc_aggregate_body(x_hbm, adj_hbm, out_hbm, src_v, dst_v, msg_v, agg,
                       gsems, ssems, slsem):
    cid = lax.axis_index("c")
    sid = lax.axis_index("s")
    wid = sid * NC + cid

    n_chunks = adj_hbm.shape[1]
    n_blocks = n_chunks // SB
    base_blocks = n_blocks // NW
    rem_blocks = n_blocks - NW * base_blocks
    start = (wid * base_blocks + jnp.minimum(wid, rem_blocks)) * SB
    k_w = (base_blocks + jnp.where(wid < rem_blocks, 1, 0)) * SB

    def zrow(r, carry):
        for col in range(128 // 16):
            msg_v[0, r, pl.ds(col * 16, 16)] = jnp.zeros((16,), jnp.float32)
        return carry

    lax.fori_loop(0, CHUNK, zrow, 0)

    def zcopy(j, carry):
        pltpu.sync_copy(msg_v.at[0], agg.at[pl.ds(sid * STRIPE + j * CHUNK,
                                                  CHUNK)])
        return carry

    lax.fori_loop(0, STRIPE // CHUNK, zcopy, 0)
    plsc.subcore_barrier()

    def slab_load(j, slot):
        off = start + j * SB
        pltpu.sync_copy(adj_hbm.at[0, pl.ds(off, SB)], src_v.at[slot])
        pltpu.sync_copy(adj_hbm.at[1, pl.ds(off, SB)], dst_v.at[slot])

    def fire_slab(j, slot):
        off = start + j * SB
        pltpu.async_copy(adj_hbm.at[0, pl.ds(off, SB)], src_v.at[slot], slsem)
        pltpu.async_copy(adj_hbm.at[1, pl.ds(off, SB)], dst_v.at[slot], slsem)

    def wait_slab(j, slot):
        off = start + j * SB
        pltpu.make_async_copy(adj_hbm.at[0, pl.ds(off, SB)],
                              src_v.at[slot], slsem).wait()
        pltpu.make_async_copy(adj_hbm.at[1, pl.ds(off, SB)],
                              dst_v.at[slot], slsem).wait()

    def idx_ref(ref, c):
        return ref.at[(c // SB) % 2, c % SB]

    def fire_gather(c, b):
        pltpu.async_copy(x_hbm.at[idx_ref(src_v, c)], msg_v.at[b], gsems[b])

    def wait_gather(c, b):
        pltpu.make_async_copy(x_hbm.at[idx_ref(src_v, c)],
                              msg_v.at[b], gsems[b]).wait()

    def fire_scatter(c, b):
        pltpu.async_copy(msg_v.at[b], agg.at[idx_ref(dst_v, c)],
                         ssems[b], add=True)

    def wait_scatter(c, b):
        pltpu.make_async_copy(msg_v.at[b], agg.at[idx_ref(dst_v, c)],
                              ssems[b]).wait()

    slab_load(0, 0)
    for c in range(LOOKAHEAD):
        fire_gather(c, c % NB)

    def quad(q, carry):
        for u in range(NB):
            c = q * NB + u
            b = u
            wait_gather(c, b)
            fire_scatter(c, b)

            @pl.when(c >= NB - LOOKAHEAD)
            def _():
                wait_scatter(c - (NB - LOOKAHEAD), (b + LOOKAHEAD) % NB)

            cg = c + LOOKAHEAD

            @pl.when(jnp.logical_and(c % SB == 2, c < k_w - SB))
            def _():
                jn = c // SB + 1
                fire_slab(jn, jn % 2)

            @pl.when(jnp.logical_and(cg % SB == 0, cg < k_w))
            def _():
                wait_slab(cg // SB, (cg // SB) % 2)

            @pl.when(cg < k_w)
            def _():
                fire_gather(cg, (b + LOOKAHEAD) % NB)

        return carry

    lax.fori_loop(0, k_w // NB, quad, 0)
    for i in range(1, NB - LOOKAHEAD + 1):
        wait_scatter(k_w - i, NB - i)
    plsc.subcore_barrier()

    off = cid * R_ACC + sid * STRIPE
    pltpu.sync_copy(agg.at[pl.ds(sid * STRIPE, STRIPE)],
                    out_hbm.at[pl.ds(off, STRIPE)])


def _sc_aggregate(x, adj3):
    mesh = plsc.VectorSubcoreMesh(core_axis_name="c", subcore_axis_name="s",
                                  num_cores=NC, num_subcores=NS)
    return pl.kernel(
        _sc_aggregate_body,
        out_type=jax.ShapeDtypeStruct((NC * R_ACC, 128), jnp.float32),
        mesh=mesh,
        scratch_types=[
            pltpu.VMEM((2, SB, CHUNK), jnp.int32),
            pltpu.VMEM((2, SB, CHUNK), jnp.int32),
            pltpu.VMEM((NB, CHUNK, 128), jnp.float32),
            pltpu.VMEM_SHARED((R_ACC, 128), jnp.float32),
            [pltpu.SemaphoreType.DMA] * NB,
            [pltpu.SemaphoreType.DMA] * NB,
            pltpu.SemaphoreType.DMA,
        ],
    )(x, adj3)


def _tc_head_body(p_ref, w_ref, b_ref, mw_ref, mb_ref, o_ref):
    s = p_ref[0] + p_ref[1]
    h = jnp.dot(s, w_ref[...], preferred_element_type=jnp.float32,
                precision=lax.Precision.HIGHEST)
    h = jnp.maximum(h + b_ref[...], 0.0)
    o = lax.dot_general(h, mw_ref[...], (((1,), (1,)), ((), ())),
                        preferred_element_type=jnp.float32,
                        precision=lax.Precision.HIGHEST)
    o_ref[...] = o + mb_ref[...]


def _tc_head(partials, W, b, mlp_W, mlp_b):
    blk = 2000
    grid = (N_NODES // blk,)
    return pl.pallas_call(
        _tc_head_body,
        grid=grid,
        in_specs=[
            pl.BlockSpec((2, blk, 128), lambda i: (0, i, 0)),
            pl.BlockSpec((128, 128), lambda i: (0, 0)),
            pl.BlockSpec((1, 128), lambda i: (0, 0)),
            pl.BlockSpec((64, 128), lambda i: (0, 0)),
            pl.BlockSpec((1, 64), lambda i: (0, 0)),
        ],
        out_specs=pl.BlockSpec((blk, 64), lambda i: (i, 0)),
        out_shape=jax.ShapeDtypeStruct((N_NODES, 64), jnp.float32),
    )(partials, W, b, mlp_W, mlp_b)


def kernel(x, adj, W, b, mlp_W, mlp_b):
    e = adj.shape[1]
    adj3 = adj.reshape(2, e // CHUNK, CHUNK)
    partials = _sc_aggregate(x, adj3)
    partials = partials.reshape(NC, R_ACC, 128)
    return _tc_head(partials, W, b.reshape(1, 128), mlp_W, mlp_b.reshape(1, 64))


# speedup vs baseline: 1.3508x; 1.0015x over previous
import jax
import jax.numpy as jnp
from jax import lax
from jax.experimental import pallas as pl
from jax.experimental.pallas import tpu as pltpu
from jax.experimental.pallas import tpu_sc as plsc

N_NODES = 10000
R_ACC = 10240
STRIPE = R_ACC // 16
CHUNK = 64
NC, NS = 2, 16
NW = NC * NS
SB = 8
NB = 4
LOOKAHEAD = 3


def _sc_aggregate_body(x_hbm, adj_hbm, out_hbm, src_v, dst_v, msg_v, agg,
                       gsems, ssems, slsem):
    cid = lax.axis_index("c")
    sid = lax.axis_index("s")
    wid = sid * NC + cid

    n_chunks = adj_hbm.shape[1]
    n_blocks = n_chunks // SB
    base_blocks = n_blocks // NW
    rem_blocks = n_blocks - NW * base_blocks
    start = (wid * base_blocks + jnp.minimum(wid, rem_blocks)) * SB
    k_w = (base_blocks + jnp.where(wid < rem_blocks, 1, 0)) * SB

    def zrow(r, carry):
        for col in range(128 // 16):
            msg_v[0, r, pl.ds(col * 16, 16)] = jnp.zeros((16,), jnp.float32)
        return carry

    lax.fori_loop(0, CHUNK, zrow, 0)

    def zcopy(j, carry):
        pltpu.async_copy(msg_v.at[0],
                         agg.at[pl.ds(sid * STRIPE + j * CHUNK, CHUNK)],
                         slsem)
        return carry

    lax.fori_loop(0, STRIPE // CHUNK, zcopy, 0)

    def zdrain(j, carry):
        pltpu.make_async_copy(msg_v.at[0],
                              agg.at[pl.ds(sid * STRIPE + j * CHUNK, CHUNK)],
                              slsem).wait()
        return carry

    lax.fori_loop(0, STRIPE // CHUNK, zdrain, 0)
    plsc.subcore_barrier()

    def slab_load(j, slot):
        off = start + j * SB
        pltpu.sync_copy(adj_hbm.at[0, pl.ds(off, SB)], src_v.at[slot])
        pltpu.sync_copy(adj_hbm.at[1, pl.ds(off, SB)], dst_v.at[slot])

    def fire_slab(j, slot):
        off = start + j * SB
        pltpu.async_copy(adj_hbm.at[0, pl.ds(off, SB)], src_v.at[slot], slsem)
        pltpu.async_copy(adj_hbm.at[1, pl.ds(off, SB)], dst_v.at[slot], slsem)

    def wait_slab(j, slot):
        off = start + j * SB
        pltpu.make_async_copy(adj_hbm.at[0, pl.ds(off, SB)],
                              src_v.at[slot], slsem).wait()
        pltpu.make_async_copy(adj_hbm.at[1, pl.ds(off, SB)],
                              dst_v.at[slot], slsem).wait()

    def idx_ref(ref, c):
        return ref.at[(c // SB) % 2, c % SB]

    def fire_gather(c, b):
        pltpu.async_copy(x_hbm.at[idx_ref(src_v, c)], msg_v.at[b], gsems[b])

    def wait_gather(c, b):
        pltpu.make_async_copy(x_hbm.at[idx_ref(src_v, c)],
                              msg_v.at[b], gsems[b]).wait()

    def fire_scatter(c, b):
        pltpu.async_copy(msg_v.at[b], agg.at[idx_ref(dst_v, c)],
                         ssems[b], add=True)

    def wait_scatter(c, b):
        pltpu.make_async_copy(msg_v.at[b], agg.at[idx_ref(dst_v, c)],
                              ssems[b]).wait()

    slab_load(0, 0)
    for c in range(LOOKAHEAD):
        fire_gather(c, c % NB)

    def quad(q, carry):
        for u in range(NB):
            c = q * NB + u
            b = u
            wait_gather(c, b)
            fire_scatter(c, b)

            @pl.when(c >= NB - LOOKAHEAD)
            def _():
                wait_scatter(c - (NB - LOOKAHEAD), (b + LOOKAHEAD) % NB)

            cg = c + LOOKAHEAD

            @pl.when(jnp.logical_and(c % SB == 2, c < k_w - SB))
            def _():
                jn = c // SB + 1
                fire_slab(jn, jn % 2)

            @pl.when(jnp.logical_and(cg % SB == 0, cg < k_w))
            def _():
                wait_slab(cg // SB, (cg // SB) % 2)

            @pl.when(cg < k_w)
            def _():
                fire_gather(cg, (b + LOOKAHEAD) % NB)

        return carry

    lax.fori_loop(0, k_w // NB, quad, 0)
    for i in range(1, NB - LOOKAHEAD + 1):
        wait_scatter(k_w - i, NB - i)
    plsc.subcore_barrier()

    off = cid * R_ACC + sid * STRIPE
    pltpu.sync_copy(agg.at[pl.ds(sid * STRIPE, STRIPE)],
                    out_hbm.at[pl.ds(off, STRIPE)])


def _sc_aggregate(x, adj3):
    mesh = plsc.VectorSubcoreMesh(core_axis_name="c", subcore_axis_name="s",
                                  num_cores=NC, num_subcores=NS)
    return pl.kernel(
        _sc_aggregate_body,
        out_type=jax.ShapeDtypeStruct((NC * R_ACC, 128), jnp.float32),
        mesh=mesh,
        scratch_types=[
            pltpu.VMEM((2, SB, CHUNK), jnp.int32),
            pltpu.VMEM((2, SB, CHUNK), jnp.int32),
            pltpu.VMEM((NB, CHUNK, 128), jnp.float32),
            pltpu.VMEM_SHARED((R_ACC, 128), jnp.float32),
            [pltpu.SemaphoreType.DMA] * NB,
            [pltpu.SemaphoreType.DMA] * NB,
            pltpu.SemaphoreType.DMA,
        ],
    )(x, adj3)


def _tc_head_body(p_ref, w_ref, b_ref, mw_ref, mb_ref, o_ref):
    s = p_ref[0] + p_ref[1]
    h = jnp.dot(s, w_ref[...], preferred_element_type=jnp.float32,
                precision=lax.Precision.HIGHEST)
    h = jnp.maximum(h + b_ref[...], 0.0)
    o = lax.dot_general(h, mw_ref[...], (((1,), (1,)), ((), ())),
                        preferred_element_type=jnp.float32,
                        precision=lax.Precision.HIGHEST)
    o_ref[...] = o + mb_ref[...]


def _tc_head(partials, W, b, mlp_W, mlp_b):
    blk = 2000
    grid = (N_NODES // blk,)
    return pl.pallas_call(
        _tc_head_body,
        grid=grid,
        in_specs=[
            pl.BlockSpec((2, blk, 128), lambda i: (0, i, 0)),
            pl.BlockSpec((128, 128), lambda i: (0, 0)),
            pl.BlockSpec((1, 128), lambda i: (0, 0)),
            pl.BlockSpec((64, 128), lambda i: (0, 0)),
            pl.BlockSpec((1, 64), lambda i: (0, 0)),
        ],
        out_specs=pl.BlockSpec((blk, 64), lambda i: (i, 0)),
        out_shape=jax.ShapeDtypeStruct((N_NODES, 64), jnp.float32),
    )(partials, W, b, mlp_W, mlp_b)


def kernel(x, adj, W, b, mlp_W, mlp_b):
    e = adj.shape[1]
    adj3 = adj.reshape(2, e // CHUNK, CHUNK)
    partials = _sc_aggregate(x, adj3)
    partials = partials.reshape(NC, R_ACC, 128)
    return _tc_head(partials, W, b.reshape(1, 128), mlp_W, mlp_b.reshape(1, 64))
